# baseline (speedup 1.0000x reference)
import jax
import jax.numpy as jnp
from jax import lax
from jax.experimental import pallas as pl
from jax.experimental.pallas import tpu as pltpu
from jax.experimental.pallas import tpu_sc as plsc

N = 10000
E = 320000
D = 128
DH = 64
NREL = 8
NW_ALL = 9

NC = 2
NS = 16

B = 128
K = 2
EPAD = 327680
ROWS = EPAD // B
DESC = ROWS // (NS * K)

NPAD = 10112
TPS = NPAD // NS

BN = 400
NBLK = N // BN


def _mm_body(f_ref, w_ref, o_ref):
    o_ref[0] = jnp.dot(f_ref[...], w_ref[0], preferred_element_type=jnp.float32)


def _compute_xw(feat, w_all):
    return pl.pallas_call(
        _mm_body,
        grid=(NW_ALL, NBLK),
        in_specs=[
            pl.BlockSpec((BN, D), lambda r, n: (n, 0)),
            pl.BlockSpec((1, D, D), lambda r, n: (r, 0, 0)),
        ],
        out_specs=pl.BlockSpec((1, BN, D), lambda r, n: (r, n, 0)),
        out_shape=jax.ShapeDtypeStruct((NW_ALL, N, D), jnp.float32),
    )(feat, w_all)


def _sc_body(xw, gh, dsth, zmh, pm,
             gv, dstv, r0, r1, g0, g1,
             msg_acc):
    rows = (r0, r1)
    gsem = (g0, g1)
    c = lax.axis_index("c")
    s = lax.axis_index("s")

    pltpu.sync_copy(gh.at[pl.ds(s * DESC * K * B, DESC * K * B)], gv)
    pltpu.sync_copy(dsth.at[pl.ds(s * DESC * K * B, DESC * K * B)], dstv)

    pltpu.sync_copy(zmh, msg_acc.at[pl.ds(s * TPS, TPS)])

    def g_vec(k, _):
        sl = pl.ds(k * 16, 16)
        gv[sl] = gv[sl] * 2 + c
        return 0
    lax.fori_loop(0, DESC * K * B // 16, g_vec, 0)

    plsc.subcore_barrier()

    def outer(i, _):
        for b in range(2):
            j = i * 2 + b
            jc = j - 1

            @pl.when(j < DESC)
            def _g(j=j, b=b):
                pltpu.async_copy(xw.at[gv.at[pl.ds(j * K * B, K * B)]],
                                 rows[b], gsem[b])

            @pl.when(jnp.logical_and(jc >= 0, jc < DESC))
            def _c(jc=jc, bc=(b + 1) % 2):
                pltpu.make_async_copy(xw.at[gv.at[pl.ds(jc * K * B, K * B)]],
                                      rows[bc], gsem[bc]).wait()
                pltpu.sync_copy(rows[bc],
                                msg_acc.at[dstv.at[pl.ds(jc * K * B, K * B)]],
                                add=True)
        return 0
    lax.fori_loop(0, DESC // 2 + 1, outer, 0)

    plsc.subcore_barrier()

    pltpu.sync_copy(msg_acc.at[pl.ds(s * TPS, TPS)], pm.at[c, pl.ds(s * TPS, TPS)])


def _sc_scatter(xw_flat, g3, dst3, zm):
    mesh = plsc.VectorSubcoreMesh(core_axis_name="c", subcore_axis_name="s")
    return pl.kernel(
        _sc_body,
        out_type=jax.ShapeDtypeStruct((NC, NPAD, DH), jnp.float32),
        mesh=mesh,
        compiler_params=pltpu.CompilerParams(use_tc_tiling_on_sc=False),
        scratch_types=[
            pltpu.VMEM((DESC * K * B,), jnp.int32),
            pltpu.VMEM((DESC * K * B,), jnp.int32),
            pltpu.VMEM((K * B, DH), jnp.float32),
            pltpu.VMEM((K * B, DH), jnp.float32),
            pltpu.SemaphoreType.DMA,
            pltpu.SemaphoreType.DMA,
            pltpu.VMEM_SHARED((NPAD, DH), jnp.float32),
        ],
    )(xw_flat, g3, dst3, zm)


EPW = EPAD // (NC * NS)
DDE = EPW // (K * B)


def _deg_body(dsth, onesh, zdh, pd, dstv, onesv, deg_acc):
    c = lax.axis_index("c")
    s = lax.axis_index("s")
    wid = c * NS + s

    pltpu.sync_copy(dsth.at[pl.ds(wid * EPW, EPW)], dstv)
    pltpu.sync_copy(onesh, onesv)
    pltpu.sync_copy(zdh, deg_acc.at[pl.ds(s * TPS, TPS)])
    plsc.subcore_barrier()

    def blk(j, _):
        pltpu.sync_copy(onesv,
                        deg_acc.at[dstv.at[pl.ds(j * K * B, K * B)]],
                        add=True)
        return 0
    lax.fori_loop(0, DDE, blk, 0)

    plsc.subcore_barrier()
    pltpu.sync_copy(deg_acc.at[pl.ds(s * TPS, TPS)], pd.at[c, pl.ds(s * TPS, TPS)])


def _sc_deg(dst3, ones, zd):
    mesh = plsc.VectorSubcoreMesh(core_axis_name="c", subcore_axis_name="s")
    return pl.kernel(
        _deg_body,
        out_type=jax.ShapeDtypeStruct((NC, NPAD, 16), jnp.float32),
        mesh=mesh,
        compiler_params=pltpu.CompilerParams(use_tc_tiling_on_sc=False),
        scratch_types=[
            pltpu.VMEM((EPW,), jnp.int32),
            pltpu.VMEM((K * B, 16), jnp.float32),
            pltpu.VMEM_SHARED((NPAD, 16), jnp.float32),
        ],
    )(dst3, ones, zd)


def _comb_body(f_ref, ce_ref, pm_ref, pd_ref, o_ref):
    f = f_ref[...]
    p = jnp.concatenate([pm_ref[0], pm_ref[1]], axis=1)
    ce = ce_ref[0]
    d = jnp.max(pd_ref[0] + pd_ref[1], axis=1, keepdims=True)
    alpha = 1.0 / jnp.maximum(d, 1.0)
    h = p * alpha + jnp.where(d > 0.0, ce, 0.0)
    o_ref[:, :D] = f
    o_ref[:, D:] = h


def _combine(feat, xw2, pm, pd):
    return pl.pallas_call(
        _comb_body,
        grid=(NBLK,),
        in_specs=[
            pl.BlockSpec((BN, D), lambda n: (n, 0)),
            pl.BlockSpec((1, BN, D), lambda n: (NREL, n, 0)),
            pl.BlockSpec((NC, BN, DH), lambda n: (0, n, 0)),
            pl.BlockSpec((NC, BN, 16), lambda n: (0, n, 0)),
        ],
        out_specs=pl.BlockSpec((BN, 2 * D), lambda n: (n, 0)),
        out_shape=jax.ShapeDtypeStruct((N, 2 * D), jnp.float32),
    )(feat, xw2, pm, pd)


def kernel(feat, edge_index, edge_type, weight, w_comp, self_loop_weight):
    rel_weight = jnp.matmul(
        w_comp, weight.reshape(weight.shape[0], -1)
    ).reshape(NREL, D, D)
    w_all = jnp.concatenate([rel_weight, self_loop_weight[None]], axis=0)

    xw2 = _compute_xw(feat, w_all)
    xw_flat = xw2.reshape(NC * NW_ALL * N, DH)

    src = edge_index[0].astype(jnp.int32)
    dst = edge_index[1].astype(jnp.int32)
    typ = edge_type.astype(jnp.int32)
    pad = EPAD - E
    g0 = typ * N + src
    g3 = jnp.concatenate([g0, jnp.zeros((pad,), jnp.int32)])
    dst3 = jnp.concatenate([dst, jnp.full((pad,), N, jnp.int32)])

    ones = jnp.ones((K * B, 16), jnp.float32)
    zm = jnp.zeros((TPS, DH), jnp.float32)
    zd = jnp.zeros((TPS, 16), jnp.float32)

    pd = _sc_deg(dst3, ones, zd)
    pm = _sc_scatter(xw_flat, g3, dst3, zm)

    return _combine(feat, xw2, pm, pd)


# speedup vs baseline: 1.2490x; 1.2490x over previous
import jax
import jax.numpy as jnp
from jax import lax
from jax.experimental import pallas as pl
from jax.experimental.pallas import tpu as pltpu
from jax.experimental.pallas import tpu_sc as plsc

N = 10000
E = 320000
D = 128
DH = 64
NREL = 8
NW_ALL = 9

NC = 2
NS = 16

B = 128
K = 1
EPAD = 327680
ROWS = EPAD // B
DESC = ROWS // (NS * K)

NPAD = 10112
TPS = NPAD // NS

BN = 400
NBLK = N // BN


def _mm_body(f_ref, w_ref, o_ref):
    f = f_ref[...]
    for r in range(NW_ALL):
        o_ref[r] = jnp.dot(f, w_ref[r], preferred_element_type=jnp.float32)


def _compute_xw(feat, w_all):
    return pl.pallas_call(
        _mm_body,
        grid=(NBLK,),
        in_specs=[
            pl.BlockSpec((BN, D), lambda n: (n, 0)),
            pl.BlockSpec((NW_ALL, D, D), lambda n: (0, 0, 0)),
        ],
        out_specs=pl.BlockSpec((NW_ALL, BN, D), lambda n: (0, n, 0)),
        out_shape=jax.ShapeDtypeStruct((NW_ALL, N, D), jnp.float32),
    )(feat, w_all)


def _sc_body(xw, gh, dsth, zmh, pm,
             gv, dstv, r0, r1, r2, g0, g1, g2, s0, s1, s2,
             msg_acc):
    rows = (r0, r1, r2)
    gsem = (g0, g1, g2)
    ssem = (s0, s1, s2)
    c = lax.axis_index("c")
    s = lax.axis_index("s")

    pltpu.sync_copy(gh.at[pl.ds(s * DESC * K * B, DESC * K * B)], gv)
    pltpu.sync_copy(dsth.at[pl.ds(s * DESC * K * B, DESC * K * B)], dstv)

    pltpu.sync_copy(zmh, msg_acc.at[pl.ds(s * TPS, TPS)])

    def g_vec(k, _):
        sl = pl.ds(k * 16, 16)
        gv[sl] = gv[sl] * 2 + c
        return 0
    lax.fori_loop(0, DESC * K * B // 16, g_vec, 0)

    plsc.subcore_barrier()

    def outer(i, _):
        for b in range(3):
            j = i * 3 + b
            jc = j - 2
            jd = j - 3

            @pl.when(jnp.logical_and(jd >= 0, jd < DESC))
            def _d(jd=jd, b=b):
                pltpu.make_async_copy(
                    rows[b],
                    msg_acc.at[dstv.at[pl.ds(jd * K * B, K * B)]],
                    ssem[b]).wait()

            @pl.when(j < DESC)
            def _g(j=j, b=b):
                pltpu.async_copy(xw.at[gv.at[pl.ds(j * K * B, K * B)]],
                                 rows[b], gsem[b])

            @pl.when(jnp.logical_and(jc >= 0, jc < DESC))
            def _c(jc=jc, bc=(b + 1) % 3):
                pltpu.make_async_copy(xw.at[gv.at[pl.ds(jc * K * B, K * B)]],
                                      rows[bc], gsem[bc]).wait()
                pltpu.async_copy(
                    rows[bc],
                    msg_acc.at[dstv.at[pl.ds(jc * K * B, K * B)]],
                    ssem[bc], add=True)
        return 0
    lax.fori_loop(0, (DESC + 3 + 2) // 3 + 1, outer, 0)

    plsc.subcore_barrier()

    pltpu.sync_copy(msg_acc.at[pl.ds(s * TPS, TPS)], pm.at[c, pl.ds(s * TPS, TPS)])


def _sc_scatter(xw_flat, g3, dst3, zm):
    mesh = plsc.VectorSubcoreMesh(core_axis_name="c", subcore_axis_name="s")
    return pl.kernel(
        _sc_body,
        out_type=jax.ShapeDtypeStruct((NC, NPAD, DH), jnp.float32),
        mesh=mesh,
        compiler_params=pltpu.CompilerParams(use_tc_tiling_on_sc=False),
        scratch_types=[
            pltpu.VMEM((DESC * K * B,), jnp.int32),
            pltpu.VMEM((DESC * K * B,), jnp.int32),
            pltpu.VMEM((K * B, DH), jnp.float32),
            pltpu.VMEM((K * B, DH), jnp.float32),
            pltpu.VMEM((K * B, DH), jnp.float32),
            pltpu.SemaphoreType.DMA,
            pltpu.SemaphoreType.DMA,
            pltpu.SemaphoreType.DMA,
            pltpu.SemaphoreType.DMA,
            pltpu.SemaphoreType.DMA,
            pltpu.SemaphoreType.DMA,
            pltpu.VMEM_SHARED((NPAD, DH), jnp.float32),
        ],
    )(xw_flat, g3, dst3, zm)


EPW = EPAD // (NC * NS)
DDE = EPW // (K * B)


def _deg_body(dsth, onesh, zdh, pd, dstv, onesv, deg_acc):
    c = lax.axis_index("c")
    s = lax.axis_index("s")
    wid = c * NS + s

    pltpu.sync_copy(dsth.at[pl.ds(wid * EPW, EPW)], dstv)
    pltpu.sync_copy(onesh, onesv)
    pltpu.sync_copy(zdh, deg_acc.at[pl.ds(s * TPS, TPS)])
    plsc.subcore_barrier()

    def blk(j, _):
        pltpu.sync_copy(onesv,
                        deg_acc.at[dstv.at[pl.ds(j * K * B, K * B)]],
                        add=True)
        return 0
    lax.fori_loop(0, DDE, blk, 0)

    plsc.subcore_barrier()
    pltpu.sync_copy(deg_acc.at[pl.ds(s * TPS, TPS)], pd.at[c, pl.ds(s * TPS, TPS)])


def _sc_deg(dst3, ones, zd):
    mesh = plsc.VectorSubcoreMesh(core_axis_name="c", subcore_axis_name="s")
    return pl.kernel(
        _deg_body,
        out_type=jax.ShapeDtypeStruct((NC, NPAD, 16), jnp.float32),
        mesh=mesh,
        compiler_params=pltpu.CompilerParams(use_tc_tiling_on_sc=False),
        scratch_types=[
            pltpu.VMEM((EPW,), jnp.int32),
            pltpu.VMEM((K * B, 16), jnp.float32),
            pltpu.VMEM_SHARED((NPAD, 16), jnp.float32),
        ],
    )(dst3, ones, zd)


def _comb_body(f_ref, ce_ref, pm_ref, pd_ref, o_ref):
    f = f_ref[...]
    p = jnp.concatenate([pm_ref[0], pm_ref[1]], axis=1)
    ce = ce_ref[0]
    d = jnp.max(pd_ref[0] + pd_ref[1], axis=1, keepdims=True)
    alpha = 1.0 / jnp.maximum(d, 1.0)
    h = p * alpha + jnp.where(d > 0.0, ce, 0.0)
    o_ref[:, :D] = f
    o_ref[:, D:] = h


def _combine(feat, xw2, pm, pd):
    return pl.pallas_call(
        _comb_body,
        grid=(NBLK,),
        in_specs=[
            pl.BlockSpec((BN, D), lambda n: (n, 0)),
            pl.BlockSpec((1, BN, D), lambda n: (NREL, n, 0)),
            pl.BlockSpec((NC, BN, DH), lambda n: (0, n, 0)),
            pl.BlockSpec((NC, BN, 16), lambda n: (0, n, 0)),
        ],
        out_specs=pl.BlockSpec((BN, 2 * D), lambda n: (n, 0)),
        out_shape=jax.ShapeDtypeStruct((N, 2 * D), jnp.float32),
    )(feat, xw2, pm, pd)


def kernel(feat, edge_index, edge_type, weight, w_comp, self_loop_weight):
    rel_weight = jnp.matmul(
        w_comp, weight.reshape(weight.shape[0], -1)
    ).reshape(NREL, D, D)
    w_all = jnp.concatenate([rel_weight, self_loop_weight[None]], axis=0)

    xw2 = _compute_xw(feat, w_all)
    xw_flat = xw2.reshape(NC * NW_ALL * N, DH)

    src = edge_index[0].astype(jnp.int32)
    dst = edge_index[1].astype(jnp.int32)
    typ = edge_type.astype(jnp.int32)
    pad = EPAD - E
    g0 = typ * N + src
    g3 = jnp.concatenate([g0, jnp.zeros((pad,), jnp.int32)])
    dst3 = jnp.concatenate([dst, jnp.full((pad,), N, jnp.int32)])

    ones = jnp.ones((K * B, 16), jnp.float32)
    zm = jnp.zeros((TPS, DH), jnp.float32)
    zd = jnp.zeros((TPS, 16), jnp.float32)

    pd = _sc_deg(dst3, ones, zd)
    pm = _sc_scatter(xw_flat, g3, dst3, zm)

    return _combine(feat, xw2, pm, pd)


# speedup vs baseline: 1.2548x; 1.0046x over previous
import jax
import jax.numpy as jnp
from jax import lax
from jax.experimental import pallas as pl
from jax.experimental.pallas import tpu as pltpu
from jax.experimental.pallas import tpu_sc as plsc

N = 10000
E = 320000
D = 128
DH = 64
NREL = 8
NW_ALL = 9

NC = 2
NS = 16

B = 128
K = 1
EPAD = 327680
ROWS = EPAD // B
DESC = ROWS // (NS * K)

NPAD = 10112
TPS = NPAD // NS

BN = 400
NBLK = N // BN


def _mm_body(f_ref, w_ref, o_ref):
    f = f_ref[...]
    for r in range(NW_ALL):
        o_ref[r] = jnp.dot(f, w_ref[r], preferred_element_type=jnp.float32)


def _compute_xw(feat, w_all):
    return pl.pallas_call(
        _mm_body,
        grid=(NBLK,),
        in_specs=[
            pl.BlockSpec((BN, D), lambda n: (n, 0)),
            pl.BlockSpec((NW_ALL, D, D), lambda n: (0, 0, 0)),
        ],
        out_specs=pl.BlockSpec((NW_ALL, BN, D), lambda n: (0, n, 0)),
        out_shape=jax.ShapeDtypeStruct((NW_ALL, N, D), jnp.float32),
    )(feat, w_all)


def _sc_body(xw, gh, dsth, zmh, pm,
             gv, dstv, r0, r1, r2, r3, g0, g1, g2, g3, s0, s1, s2, s3,
             msg_acc):
    rows = (r0, r1, r2, r3)
    gsem = (g0, g1, g2, g3)
    ssem = (s0, s1, s2, s3)
    c = lax.axis_index("c")
    s = lax.axis_index("s")

    pltpu.sync_copy(gh.at[pl.ds(s * DESC * K * B, DESC * K * B)], gv)
    pltpu.sync_copy(dsth.at[pl.ds(s * DESC * K * B, DESC * K * B)], dstv)

    pltpu.sync_copy(zmh, msg_acc.at[pl.ds(s * TPS, TPS)])

    def g_vec(k, _):
        sl = pl.ds(k * 16, 16)
        gv[sl] = gv[sl] * 2 + c
        return 0
    lax.fori_loop(0, DESC * K * B // 16, g_vec, 0)

    plsc.subcore_barrier()

    def outer(i, _):
        for b in range(4):
            j = i * 4 + b
            jc = j - 3
            jd = j - 4

            @pl.when(jnp.logical_and(jd >= 0, jd < DESC))
            def _d(jd=jd, b=b):
                pltpu.make_async_copy(
                    rows[b],
                    msg_acc.at[dstv.at[pl.ds(jd * K * B, K * B)]],
                    ssem[b]).wait()

            @pl.when(j < DESC)
            def _g(j=j, b=b):
                pltpu.async_copy(xw.at[gv.at[pl.ds(j * K * B, K * B)]],
                                 rows[b], gsem[b])

            @pl.when(jnp.logical_and(jc >= 0, jc < DESC))
            def _c(jc=jc, bc=(b + 1) % 4):
                pltpu.make_async_copy(xw.at[gv.at[pl.ds(jc * K * B, K * B)]],
                                      rows[bc], gsem[bc]).wait()
                pltpu.async_copy(
                    rows[bc],
                    msg_acc.at[dstv.at[pl.ds(jc * K * B, K * B)]],
                    ssem[bc], add=True)
        return 0
    lax.fori_loop(0, (DESC + 4 + 3) // 4 + 1, outer, 0)

    plsc.subcore_barrier()

    pltpu.sync_copy(msg_acc.at[pl.ds(s * TPS, TPS)], pm.at[c, pl.ds(s * TPS, TPS)])


def _sc_scatter(xw_flat, g3, dst3, zm):
    mesh = plsc.VectorSubcoreMesh(core_axis_name="c", subcore_axis_name="s")
    return pl.kernel(
        _sc_body,
        out_type=jax.ShapeDtypeStruct((NC, NPAD, DH), jnp.float32),
        mesh=mesh,
        compiler_params=pltpu.CompilerParams(use_tc_tiling_on_sc=False),
        scratch_types=[
            pltpu.VMEM((DESC * K * B,), jnp.int32),
            pltpu.VMEM((DESC * K * B,), jnp.int32),
            pltpu.VMEM((K * B, DH), jnp.float32),
            pltpu.VMEM((K * B, DH), jnp.float32),
            pltpu.VMEM((K * B, DH), jnp.float32),
            pltpu.VMEM((K * B, DH), jnp.float32),
            pltpu.SemaphoreType.DMA,
            pltpu.SemaphoreType.DMA,
            pltpu.SemaphoreType.DMA,
            pltpu.SemaphoreType.DMA,
            pltpu.SemaphoreType.DMA,
            pltpu.SemaphoreType.DMA,
            pltpu.SemaphoreType.DMA,
            pltpu.SemaphoreType.DMA,
            pltpu.VMEM_SHARED((NPAD, DH), jnp.float32),
        ],
    )(xw_flat, g3, dst3, zm)


EPW = EPAD // (NC * NS)
DDE = EPW // (K * B)


def _deg_body(dsth, onesh, zdh, pd, dstv, onesv, deg_acc):
    c = lax.axis_index("c")
    s = lax.axis_index("s")
    wid = c * NS + s

    pltpu.sync_copy(dsth.at[pl.ds(wid * EPW, EPW)], dstv)
    pltpu.sync_copy(onesh, onesv)
    pltpu.sync_copy(zdh, deg_acc.at[pl.ds(s * TPS, TPS)])
    plsc.subcore_barrier()

    def blk(j, _):
        pltpu.sync_copy(onesv,
                        deg_acc.at[dstv.at[pl.ds(j * K * B, K * B)]],
                        add=True)
        return 0
    lax.fori_loop(0, DDE, blk, 0)

    plsc.subcore_barrier()
    pltpu.sync_copy(deg_acc.at[pl.ds(s * TPS, TPS)], pd.at[c, pl.ds(s * TPS, TPS)])


def _sc_deg(dst3, ones, zd):
    mesh = plsc.VectorSubcoreMesh(core_axis_name="c", subcore_axis_name="s")
    return pl.kernel(
        _deg_body,
        out_type=jax.ShapeDtypeStruct((NC, NPAD, 16), jnp.float32),
        mesh=mesh,
        compiler_params=pltpu.CompilerParams(use_tc_tiling_on_sc=False),
        scratch_types=[
            pltpu.VMEM((EPW,), jnp.int32),
            pltpu.VMEM((K * B, 16), jnp.float32),
            pltpu.VMEM_SHARED((NPAD, 16), jnp.float32),
        ],
    )(dst3, ones, zd)


def _comb_body(f_ref, ce_ref, pm_ref, pd_ref, o_ref):
    f = f_ref[...]
    p = jnp.concatenate([pm_ref[0], pm_ref[1]], axis=1)
    ce = ce_ref[0]
    d = jnp.max(pd_ref[0] + pd_ref[1], axis=1, keepdims=True)
    alpha = 1.0 / jnp.maximum(d, 1.0)
    h = p * alpha + jnp.where(d > 0.0, ce, 0.0)
    o_ref[:, :D] = f
    o_ref[:, D:] = h


def _combine(feat, xw2, pm, pd):
    return pl.pallas_call(
        _comb_body,
        grid=(NBLK,),
        in_specs=[
            pl.BlockSpec((BN, D), lambda n: (n, 0)),
            pl.BlockSpec((1, BN, D), lambda n: (NREL, n, 0)),
            pl.BlockSpec((NC, BN, DH), lambda n: (0, n, 0)),
            pl.BlockSpec((NC, BN, 16), lambda n: (0, n, 0)),
        ],
        out_specs=pl.BlockSpec((BN, 2 * D), lambda n: (n, 0)),
        out_shape=jax.ShapeDtypeStruct((N, 2 * D), jnp.float32),
    )(feat, xw2, pm, pd)


def kernel(feat, edge_index, edge_type, weight, w_comp, self_loop_weight):
    rel_weight = jnp.matmul(
        w_comp, weight.reshape(weight.shape[0], -1)
    ).reshape(NREL, D, D)
    w_all = jnp.concatenate([rel_weight, self_loop_weight[None]], axis=0)

    xw2 = _compute_xw(feat, w_all)
    xw_flat = xw2.reshape(NC * NW_ALL * N, DH)

    src = edge_index[0].astype(jnp.int32)
    dst = edge_index[1].astype(jnp.int32)
    typ = edge_type.astype(jnp.int32)
    pad = EPAD - E
    g0 = typ * N + src
    g3 = jnp.concatenate([g0, jnp.zeros((pad,), jnp.int32)])
    dst3 = jnp.concatenate([dst, jnp.full((pad,), N, jnp.int32)])

    ones = jnp.ones((K * B, 16), jnp.float32)
    zm = jnp.zeros((TPS, DH), jnp.float32)
    zd = jnp.zeros((TPS, 16), jnp.float32)

    pd = _sc_deg(dst3, ones, zd)
    pm = _sc_scatter(xw_flat, g3, dst3, zm)

    return _combine(feat, xw2, pm, pd)
